# baseline (speedup 1.0000x reference)
import functools

import jax
import jax.numpy as jnp
from jax import lax
from jax.experimental import pallas as pl
from jax.experimental.pallas import tpu as pltpu
from jax.experimental.pallas import tpu_sc as plsc

EMBED = 32
LANES = 128
PACK = LANES // EMBED
B = 4096 * 200
NC, NS = 2, 16
NW = NC * NS
B_PER_W = B // NW
CHUNK = 1600
NCHUNK = B_PER_W // CHUNK
OROWS = CHUNK // PACK
OROWS_PER_W = B_PER_W // PACK

_mesh = plsc.VectorSubcoreMesh(core_axis_name="c", subcore_axis_name="s")


@functools.partial(
    pl.kernel,
    mesh=_mesh,
    out_type=jax.ShapeDtypeStruct((B, EMBED), jnp.float32),
    scratch_types=[
        pltpu.VMEM((B_PER_W,), jnp.int32),
        pltpu.VMEM((2, CHUNK, EMBED), jnp.float32),
        pltpu.SemaphoreType.DMA,
        pltpu.SemaphoreType.DMA,
        pltpu.SemaphoreType.DMA,
        pltpu.SemaphoreType.DMA,
    ],
    compiler_params=pltpu.CompilerParams(use_tc_tiling_on_sc=False),
)
def _gather_kernel(idx_hbm, table_hbm, out_hbm, idx_v, rows_v,
                   gsem0, gsem1, ssem0, ssem1):
    wid = lax.axis_index("s") * NC + lax.axis_index("c")
    base = wid * B_PER_W
    obase = wid * OROWS_PER_W
    gsems = (gsem0, gsem1)
    ssems = (ssem0, ssem1)
    pltpu.sync_copy(idx_hbm.at[pl.ds(base, B_PER_W)], idx_v)

    def start_gather(c, buf):
        return pltpu.async_copy(
            table_hbm.at[idx_v.at[pl.ds(c * CHUNK, CHUNK)]],
            rows_v.at[buf], gsems[buf])

    def start_writeout(c, buf):
        return pltpu.async_copy(
            rows_v.at[buf],
            out_hbm.at[pl.ds(base + c * CHUNK, CHUNK)],
            ssems[buf])

    gathers = [start_gather(0, 0), None]
    writes = [None, None]
    for c in range(NCHUNK):
        cur = c & 1
        nxt = 1 - cur
        if c + 1 < NCHUNK:
            if writes[nxt] is not None:
                writes[nxt].wait()
            gathers[nxt] = start_gather(c + 1, nxt)
        gathers[cur].wait()
        writes[cur] = start_writeout(c, cur)
    for w in writes:
        if w is not None:
            w.wait()


def kernel(sequence, table):
    batch, hist = sequence.shape
    idx = sequence.reshape(-1).astype(jnp.int32)
    out = _gather_kernel(idx, table)
    return out.reshape(batch, hist, EMBED)


# speedup vs baseline: 1.3636x; 1.3636x over previous
import functools

import jax
import jax.numpy as jnp
from jax import lax
from jax.experimental import pallas as pl
from jax.experimental.pallas import tpu as pltpu
from jax.experimental.pallas import tpu_sc as plsc

EMBED = 32
LANES = 128
B = 4096 * 200
NC, NS = 2, 16
NW = NC * NS
B_PER_W = B // NW
CHUNK = 1600
NCHUNK = B_PER_W // CHUNK

_mesh = plsc.VectorSubcoreMesh(core_axis_name="c", subcore_axis_name="s")


@functools.partial(
    pl.kernel,
    mesh=_mesh,
    out_type=jax.ShapeDtypeStruct((B, LANES), jnp.float32),
    scratch_types=[
        pltpu.VMEM((B_PER_W,), jnp.int32),
        pltpu.VMEM((2, CHUNK, EMBED), jnp.float32),
        pltpu.SemaphoreType.DMA,
        pltpu.SemaphoreType.DMA,
        pltpu.SemaphoreType.DMA,
        pltpu.SemaphoreType.DMA,
    ],
    compiler_params=pltpu.CompilerParams(use_tc_tiling_on_sc=False),
)
def _gather_kernel(idx_hbm, table_hbm, out_hbm, idx_v, rows_v,
                   gsem0, gsem1, ssem0, ssem1):
    wid = lax.axis_index("s") * NC + lax.axis_index("c")
    base = wid * B_PER_W
    gsems = (gsem0, gsem1)
    ssems = (ssem0, ssem1)
    pltpu.sync_copy(idx_hbm.at[pl.ds(base, B_PER_W)], idx_v)

    def start_gather(c, buf):
        return pltpu.async_copy(
            table_hbm.at[idx_v.at[pl.ds(c * CHUNK, CHUNK)]],
            rows_v.at[buf], gsems[buf])

    def start_writeout(c, buf):
        return pltpu.async_copy(
            rows_v.at[buf],
            out_hbm.at[pl.ds(base + c * CHUNK, CHUNK), pl.ds(0, EMBED)],
            ssems[buf])

    gathers = [start_gather(0, 0), None]
    writes = [None, None]
    for c in range(NCHUNK):
        cur = c & 1
        nxt = 1 - cur
        if c + 1 < NCHUNK:
            if writes[nxt] is not None:
                writes[nxt].wait()
            gathers[nxt] = start_gather(c + 1, nxt)
        gathers[cur].wait()
        writes[cur] = start_writeout(c, cur)
    for w in writes:
        if w is not None:
            w.wait()


def kernel(sequence, table):
    batch, hist = sequence.shape
    idx = sequence.reshape(-1).astype(jnp.int32)
    out = _gather_kernel(idx, table)
    return out[:, :EMBED].reshape(batch, hist, EMBED)
